# Initial kernel scaffold; baseline (speedup 1.0000x reference)
import jax
import jax.numpy as jnp
from jax.experimental import pallas as pl


def kernel(x, edge_index, edge_weight, batch, conv_W, conv_b, lin1_W, lin1_b, lin2_W, lin2_b):
    raise NotImplementedError("write your pallas kernel here")


# speedup vs baseline: 1.0469x; 1.0469x over previous
import functools
import jax
import jax.numpy as jnp
from jax import lax
from jax.experimental import pallas as pl
from jax.experimental.pallas import tpu as pltpu

N = 10000
E = 320000
D = 128
K = 3
L = 5
G = 64

NB = 10
BN = N // NB


def _layer_body(h_ref, p1_ref, p2_ref, p3_ref, W_ref, b_ref, o_ref):
    acc = jnp.dot(h_ref[...], W_ref[0], preferred_element_type=jnp.float32)
    acc += jnp.dot(p1_ref[...], W_ref[1], preferred_element_type=jnp.float32)
    acc += jnp.dot(p2_ref[...], W_ref[2], preferred_element_type=jnp.float32)
    acc += jnp.dot(p3_ref[...], W_ref[3], preferred_element_type=jnp.float32)
    o_ref[...] = jnp.maximum(acc + b_ref[...], 0.0)


def _layer_matmul(h, p1, p2, p3, W, bsum):
    grid = (NB,)
    blk = lambda i: (i, 0)
    return pl.pallas_call(
        _layer_body,
        grid=grid,
        in_specs=[
            pl.BlockSpec((BN, D), blk),
            pl.BlockSpec((BN, D), blk),
            pl.BlockSpec((BN, D), blk),
            pl.BlockSpec((BN, D), blk),
            pl.BlockSpec((K + 1, D, D), lambda i: (0, 0, 0)),
            pl.BlockSpec((1, D), lambda i: (0, 0)),
        ],
        out_specs=pl.BlockSpec((BN, D), blk),
        out_shape=jax.ShapeDtypeStruct((N, D), jnp.float32),
    )(h, p1, p2, p3, W, bsum)


def _head_body(h_ref, batch_ref, l1w_ref, l1b_ref, l2w_ref, l2b_ref, o_ref):
    batch = batch_ref[...]
    gids = lax.broadcasted_iota(jnp.int32, (G, N), 0)
    sel = (gids == batch).astype(jnp.float32)
    sums = jnp.dot(sel, h_ref[...], preferred_element_type=jnp.float32)
    cnt = jnp.sum(sel, axis=1, keepdims=True)
    pooled = sums / jnp.maximum(cnt, 1.0)
    y = jnp.dot(pooled, l1w_ref[...], preferred_element_type=jnp.float32)
    y = y + l1b_ref[...]
    y = jnp.dot(y, l2w_ref[...], preferred_element_type=jnp.float32)
    y = y + l2b_ref[...]
    o_ref[...] = jax.nn.sigmoid(y)


def _head(h, batch, lin1_W, lin1_b, lin2_W, lin2_b):
    return pl.pallas_call(
        _head_body,
        out_shape=jax.ShapeDtypeStruct((G, 1), jnp.float32),
    )(h, batch.reshape(1, N), lin1_W, lin1_b.reshape(1, D),
      lin2_W, lin2_b.reshape(1, 1))


def kernel(x, edge_index, edge_weight, batch, conv_W, conv_b, lin1_W, lin1_b,
           lin2_W, lin2_b):
    row, col = edge_index[0], edge_index[1]
    deg = jnp.zeros((N,), jnp.float32).at[col].add(edge_weight)
    dis = jnp.where(deg > 0, lax.rsqrt(deg), 0.0)
    norm = dis[row] * edge_weight * dis[col]

    bsum = jnp.sum(conv_b, axis=1).reshape(L, 1, D)

    h = x
    for i in range(L):
        p = h
        ps = []
        for k in range(K):
            msg = p[row] * norm[:, None]
            p = jnp.zeros((N, D), jnp.float32).at[col].add(msg)
            ps.append(p)
        h = _layer_matmul(h, ps[0], ps[1], ps[2], conv_W[i], bsum[i])

    return _head(h, batch, lin1_W, lin1_b, lin2_W, lin2_b)


# speedup vs baseline: 2.2390x; 2.1386x over previous
import functools
import jax
import jax.numpy as jnp
from jax import lax
from jax.experimental import pallas as pl
from jax.experimental.pallas import tpu as pltpu
from jax.experimental.pallas import tpu_sc as plsc

N = 10000
E = 320000
D = 128
K = 3
L = 5
G = 64

NSUB = 16
NCORE = 2
CHUNK = 128
EPW = E // (NCORE * NSUB)
CPT = 80
EPT = CPT * CHUNK
STG = 40
N_PAD = 10240
RPT = N_PAD // NSUB
ZR = 64

_mesh = plsc.VectorSubcoreMesh(core_axis_name="c", subcore_axis_name="s")


def _spmm_body(src_ref, ridx_hbm, cidx_hbm, nrm_hbm, out_ref,
               acc, ridx_v, cidx_v, nrm_v, gbuf, sem):
    c = lax.axis_index("c")
    s = lax.axis_index("s")
    w = c * NSUB + s

    zero16 = jnp.zeros((16,), jnp.float32)

    def _z(i, carry):
        r = i // (D // 16)
        v = i % (D // 16)
        gbuf[r, pl.ds(v * 16, 16)] = zero16
        return carry
    lax.fori_loop(0, ZR * (D // 16), _z, 0)

    base = s * RPT
    for i in range(RPT // ZR):
        pltpu.sync_copy(gbuf.at[pl.ds(0, ZR)],
                        acc.at[pl.ds(base + i * ZR, ZR)])
    plsc.subcore_barrier()

    def _chunk(j, carry):
        pltpu.async_copy(src_ref.at[ridx_v.at[j]], gbuf, sem).wait()

        def _scale(g, c2):
            nv = nrm_v[j, pl.ds(g * 16, 16)]
            for r in range(16):
                ns = nv[r]
                for v in range(D // 16):
                    sl = pl.ds(v * 16, 16)
                    gbuf[g * 16 + r, sl] = gbuf[g * 16 + r, sl] * ns
            return c2
        lax.fori_loop(0, CHUNK // 16, _scale, 0)
        pltpu.sync_copy(gbuf, acc.at[cidx_v.at[j]], add=True)
        return carry

    for half in range(CPT // STG):
        sl = pl.ds(half * STG, STG)
        pltpu.sync_copy(ridx_hbm.at[w, sl], ridx_v)
        pltpu.sync_copy(cidx_hbm.at[w, sl], cidx_v)
        pltpu.sync_copy(nrm_hbm.at[w, sl], nrm_v)
        lax.fori_loop(0, STG, _chunk, 0)
    plsc.subcore_barrier()

    pltpu.sync_copy(acc.at[pl.ds(base, RPT)],
                    out_ref.at[pl.ds(c * N_PAD + base, RPT)])


_spmm = functools.partial(
    pl.kernel,
    out_type=jax.ShapeDtypeStruct((NCORE * N_PAD, D), jnp.float32),
    mesh=_mesh,
    scratch_types=[
        pltpu.VMEM_SHARED((N_PAD, D), jnp.float32),
        pltpu.VMEM((STG, CHUNK), jnp.int32),
        pltpu.VMEM((STG, CHUNK), jnp.int32),
        pltpu.VMEM((STG, CHUNK), jnp.float32),
        pltpu.VMEM((CHUNK, D), jnp.float32),
        pltpu.SemaphoreType.DMA,
    ],
)(_spmm_body)


NB = 10
BN = N // NB


def _combine_body(pa, pb, o_ref):
    o_ref[...] = pa[0] + pb[0]


def _combine(p):
    return pl.pallas_call(
        _combine_body,
        grid=(NB,),
        in_specs=[pl.BlockSpec((1, BN, D), lambda i: (0, i, 0)),
                  pl.BlockSpec((1, BN, D), lambda i: (1, i, 0))],
        out_specs=pl.BlockSpec((BN, D), lambda i: (i, 0)),
        out_shape=jax.ShapeDtypeStruct((N_PAD, D), jnp.float32),
    )(p, p)


def _layer_body(h_ref, p1_ref, p2_ref, p3a, p3b, W_ref, b_ref, o_ref):
    p3 = p3a[0] + p3b[0]
    acc = jnp.dot(h_ref[...], W_ref[0], preferred_element_type=jnp.float32)
    acc += jnp.dot(p1_ref[...], W_ref[1], preferred_element_type=jnp.float32)
    acc += jnp.dot(p2_ref[...], W_ref[2], preferred_element_type=jnp.float32)
    acc += jnp.dot(p3, W_ref[3], preferred_element_type=jnp.float32)
    o_ref[...] = jnp.maximum(acc + b_ref[...], 0.0)


def _layer_matmul(h, p1, p2, p3, W, bsum):
    blk = lambda i: (i, 0)
    return pl.pallas_call(
        _layer_body,
        grid=(NB,),
        in_specs=[pl.BlockSpec((BN, D), blk),
                  pl.BlockSpec((BN, D), blk),
                  pl.BlockSpec((BN, D), blk),
                  pl.BlockSpec((1, BN, D), lambda i: (0, i, 0)),
                  pl.BlockSpec((1, BN, D), lambda i: (1, i, 0)),
                  pl.BlockSpec((K + 1, D, D), lambda i: (0, 0, 0)),
                  pl.BlockSpec((1, D), lambda i: (0, 0))],
        out_specs=pl.BlockSpec((BN, D), blk),
        out_shape=jax.ShapeDtypeStruct((N_PAD, D), jnp.float32),
    )(h, p1, p2, p3, p3, W, bsum)


def _head_body(h_ref, batch_ref, l1w_ref, l1b_ref, l2w_ref, l2b_ref, o_ref):
    h = h_ref[...][:N]
    batch = batch_ref[...]
    gids = lax.broadcasted_iota(jnp.int32, (G, N), 0)
    sel = (gids == batch).astype(jnp.float32)
    sums = jnp.dot(sel, h, preferred_element_type=jnp.float32)
    cnt = jnp.sum(sel, axis=1, keepdims=True)
    pooled = sums / jnp.maximum(cnt, 1.0)
    y = jnp.dot(pooled, l1w_ref[...], preferred_element_type=jnp.float32)
    y = y + l1b_ref[...]
    y = jnp.dot(y, l2w_ref[...], preferred_element_type=jnp.float32)
    y = y + l2b_ref[...]
    o_ref[...] = jax.nn.sigmoid(y)


def _head(h, batch, lin1_W, lin1_b, lin2_W, lin2_b):
    return pl.pallas_call(
        _head_body,
        out_shape=jax.ShapeDtypeStruct((G, 1), jnp.float32),
    )(h, batch.reshape(1, N), lin1_W, lin1_b.reshape(1, D),
      lin2_W, lin2_b.reshape(1, 1))


def _prep_edges(arr):
    a = jnp.zeros((NCORE * NSUB, EPT), arr.dtype)
    a = a.at[:, :EPW].set(arr.reshape(NCORE * NSUB, EPW))
    return a.reshape(NCORE * NSUB, CPT, CHUNK)


def kernel(x, edge_index, edge_weight, batch, conv_W, conv_b, lin1_W, lin1_b,
           lin2_W, lin2_b):
    row, col = edge_index[0], edge_index[1]
    deg = jnp.zeros((N,), jnp.float32).at[col].add(edge_weight)
    dis = jnp.where(deg > 0, lax.rsqrt(deg), 0.0)
    norm = dis[row] * edge_weight * dis[col]

    ridx = _prep_edges(row)
    cidx = _prep_edges(col)
    nrm = _prep_edges(norm)

    bsum = jnp.sum(conv_b, axis=1).reshape(L, 1, D)

    h = jnp.zeros((N_PAD, D), jnp.float32).at[:N].set(x)
    for i in range(L):
        src = h
        ps = []
        for k in range(K):
            part = _spmm(src, ridx, cidx, nrm).reshape(NCORE, N_PAD, D)
            ps.append(part)
            if k < K - 1:
                src = _combine(part)
                ps[k] = src
        h = _layer_matmul(h, ps[0], ps[1], ps[2], conv_W[i], bsum[i])

    return _head(h, batch, lin1_W, lin1_b, lin2_W, lin2_b)


# speedup vs baseline: 2.4716x; 1.1039x over previous
import functools
import jax
import jax.numpy as jnp
from jax import lax
from jax.experimental import pallas as pl
from jax.experimental.pallas import tpu as pltpu
from jax.experimental.pallas import tpu_sc as plsc

N = 10000
E = 320000
D = 128
K = 3
L = 5
G = 64

NSUB = 16
NCORE = 2
CHUNK = 128
EPW = E // (NCORE * NSUB)
CPT = 80
EPT = CPT * CHUNK
STG = 16
N_PAD = 10240
RPT = N_PAD // NSUB
ZR = 64

_mesh = plsc.VectorSubcoreMesh(core_axis_name="c", subcore_axis_name="s")


def _spmm_body(src_ref, ridx_hbm, cidx_hbm, nrm_hbm, out_ref,
               acc, ridx_v, cidx_v, nrm_v, gbuf, gsem):
    c = lax.axis_index("c")
    s = lax.axis_index("s")
    w = c * NSUB + s

    zero16 = jnp.zeros((16,), jnp.float32)

    def _z(i, carry):
        r = i // (D // 16)
        v = i % (D // 16)
        gbuf[0, r, pl.ds(v * 16, 16)] = zero16
        return carry
    lax.fori_loop(0, ZR * (D // 16), _z, 0)

    base = s * RPT
    for i in range(RPT // ZR):
        pltpu.sync_copy(gbuf.at[0, pl.ds(0, ZR)],
                        acc.at[pl.ds(base + i * ZR, ZR)])
    plsc.subcore_barrier()

    def _start_gather(jl, b):
        pltpu.async_copy(src_ref.at[ridx_v.at[jl]], gbuf.at[b], gsem)

    def _wait_gather(b):
        pltpu.make_async_copy(src_ref.at[ridx_v.at[0]], gbuf.at[b],
                              gsem).wait()

    def _scale(jl, b):
        def _grp(g, c2):
            nv = nrm_v[jl, pl.ds(g * 16, 16)]
            for r in range(16):
                ns = nv[r]
                for v in range(D // 16):
                    sl = pl.ds(v * 16, 16)
                    gbuf[b, g * 16 + r, sl] = gbuf[b, g * 16 + r, sl] * ns
            return c2
        lax.fori_loop(0, CHUNK // 16, _grp, 0)

    for st in range(CPT // STG):
        ssl = pl.ds(st * STG, STG)
        pltpu.sync_copy(ridx_hbm.at[w, ssl], ridx_v)
        pltpu.sync_copy(cidx_hbm.at[w, ssl], cidx_v)
        pltpu.sync_copy(nrm_hbm.at[w, ssl], nrm_v)
        _start_gather(0, 0)
        _start_gather(1, 1)

        def _pair(jj, carry):
            for b in range(2):
                jl = 2 * jj + b
                _wait_gather(b)
                _scale(jl, b)
                pltpu.sync_copy(gbuf.at[b], acc.at[cidx_v.at[jl]], add=True)

                @pl.when(jl + 2 < STG)
                def _():
                    _start_gather(jl + 2, b)
            return carry
        lax.fori_loop(0, STG // 2, _pair, 0)
    plsc.subcore_barrier()

    pltpu.sync_copy(acc.at[pl.ds(base, RPT)],
                    out_ref.at[pl.ds(c * N_PAD + base, RPT)])


_spmm = functools.partial(
    pl.kernel,
    out_type=jax.ShapeDtypeStruct((NCORE * N_PAD, D), jnp.float32),
    mesh=_mesh,
    scratch_types=[
        pltpu.VMEM_SHARED((N_PAD, D), jnp.float32),
        pltpu.VMEM((STG, CHUNK), jnp.int32),
        pltpu.VMEM((STG, CHUNK), jnp.int32),
        pltpu.VMEM((STG, CHUNK), jnp.float32),
        pltpu.VMEM((2, CHUNK, D), jnp.float32),
        pltpu.SemaphoreType.DMA,
    ],
)(_spmm_body)


NB = 10
BN = N // NB


def _combine_body(pa, pb, o_ref):
    o_ref[...] = pa[0] + pb[0]


def _combine(p):
    return pl.pallas_call(
        _combine_body,
        grid=(NB,),
        in_specs=[pl.BlockSpec((1, BN, D), lambda i: (0, i, 0)),
                  pl.BlockSpec((1, BN, D), lambda i: (1, i, 0))],
        out_specs=pl.BlockSpec((BN, D), lambda i: (i, 0)),
        out_shape=jax.ShapeDtypeStruct((N_PAD, D), jnp.float32),
    )(p, p)


def _layer_body(h_ref, p1_ref, p2_ref, p3a, p3b, W_ref, b_ref, o_ref):
    p3 = p3a[0] + p3b[0]
    acc = jnp.dot(h_ref[...], W_ref[0], preferred_element_type=jnp.float32)
    acc += jnp.dot(p1_ref[...], W_ref[1], preferred_element_type=jnp.float32)
    acc += jnp.dot(p2_ref[...], W_ref[2], preferred_element_type=jnp.float32)
    acc += jnp.dot(p3, W_ref[3], preferred_element_type=jnp.float32)
    o_ref[...] = jnp.maximum(acc + b_ref[...], 0.0)


def _layer_matmul(h, p1, p2, p3, W, bsum):
    blk = lambda i: (i, 0)
    return pl.pallas_call(
        _layer_body,
        grid=(NB,),
        in_specs=[pl.BlockSpec((BN, D), blk),
                  pl.BlockSpec((BN, D), blk),
                  pl.BlockSpec((BN, D), blk),
                  pl.BlockSpec((1, BN, D), lambda i: (0, i, 0)),
                  pl.BlockSpec((1, BN, D), lambda i: (1, i, 0)),
                  pl.BlockSpec((K + 1, D, D), lambda i: (0, 0, 0)),
                  pl.BlockSpec((1, D), lambda i: (0, 0))],
        out_specs=pl.BlockSpec((BN, D), blk),
        out_shape=jax.ShapeDtypeStruct((N_PAD, D), jnp.float32),
    )(h, p1, p2, p3, p3, W, bsum)


def _head_body(h_ref, batch_ref, l1w_ref, l1b_ref, l2w_ref, l2b_ref, o_ref):
    h = h_ref[...][:N]
    batch = batch_ref[...]
    gids = lax.broadcasted_iota(jnp.int32, (G, N), 0)
    sel = (gids == batch).astype(jnp.float32)
    sums = jnp.dot(sel, h, preferred_element_type=jnp.float32)
    cnt = jnp.sum(sel, axis=1, keepdims=True)
    pooled = sums / jnp.maximum(cnt, 1.0)
    y = jnp.dot(pooled, l1w_ref[...], preferred_element_type=jnp.float32)
    y = y + l1b_ref[...]
    y = jnp.dot(y, l2w_ref[...], preferred_element_type=jnp.float32)
    y = y + l2b_ref[...]
    o_ref[...] = jax.nn.sigmoid(y)


def _head(h, batch, lin1_W, lin1_b, lin2_W, lin2_b):
    return pl.pallas_call(
        _head_body,
        out_shape=jax.ShapeDtypeStruct((G, 1), jnp.float32),
    )(h, batch.reshape(1, N), lin1_W, lin1_b.reshape(1, D),
      lin2_W, lin2_b.reshape(1, 1))


def _prep_edges(arr):
    a = jnp.zeros((NCORE * NSUB, EPT), arr.dtype)
    a = a.at[:, :EPW].set(arr.reshape(NCORE * NSUB, EPW))
    return a.reshape(NCORE * NSUB, CPT, CHUNK)


def kernel(x, edge_index, edge_weight, batch, conv_W, conv_b, lin1_W, lin1_b,
           lin2_W, lin2_b):
    row, col = edge_index[0], edge_index[1]
    deg = jnp.zeros((N,), jnp.float32).at[col].add(edge_weight)
    dis = jnp.where(deg > 0, lax.rsqrt(deg), 0.0)
    norm = dis[row] * edge_weight * dis[col]

    ridx = _prep_edges(row)
    cidx = _prep_edges(col)
    nrm = _prep_edges(norm)

    bsum = jnp.sum(conv_b, axis=1).reshape(L, 1, D)

    h = jnp.zeros((N_PAD, D), jnp.float32).at[:N].set(x)
    for i in range(L):
        src = h
        ps = []
        for k in range(K):
            part = _spmm(src, ridx, cidx, nrm).reshape(NCORE, N_PAD, D)
            ps.append(part)
            if k < K - 1:
                src = _combine(part)
                ps[k] = src
        h = _layer_matmul(h, ps[0], ps[1], ps[2], conv_W[i], bsum[i])

    return _head(h, batch, lin1_W, lin1_b, lin2_W, lin2_b)


# speedup vs baseline: 3.7806x; 1.5296x over previous
import functools
import jax
import jax.numpy as jnp
from jax import lax
from jax.experimental import pallas as pl
from jax.experimental.pallas import tpu as pltpu
from jax.experimental.pallas import tpu_sc as plsc

N = 10000
E = 320000
D = 128
K = 3
L = 5
G = 64

NSUB = 16
NCORE = 2
NW = NCORE * NSUB
CHUNK = 128
EPW = E // NW
CPT = 80
EPT = CPT * CHUNK
STG = 16
N_PAD = 10240
RPT = N_PAD // NSUB
ZR = 64

_mesh = plsc.VectorSubcoreMesh(core_axis_name="c", subcore_axis_name="s")


def _spmm_body(src_ref, ridx_hbm, cidx_hbm, ew_hbm, out_ref,
               acc, ridx_v, cidx_v, ew_v, gbuf, gsem):
    c = lax.axis_index("c")
    s = lax.axis_index("s")
    w = c * NSUB + s

    zero16 = jnp.zeros((16,), jnp.float32)

    def _z(i, carry):
        r = i // (D // 16)
        v = i % (D // 16)
        gbuf[0, r, pl.ds(v * 16, 16)] = zero16
        return carry
    lax.fori_loop(0, ZR * (D // 16), _z, 0)

    base = s * RPT
    for i in range(RPT // ZR):
        pltpu.sync_copy(gbuf.at[0, pl.ds(0, ZR)],
                        acc.at[pl.ds(base + i * ZR, ZR)])
    plsc.subcore_barrier()

    def _start_gather(jl, b):
        pltpu.async_copy(src_ref.at[ridx_v.at[jl]], gbuf.at[b], gsem)

    def _wait_gather(b):
        pltpu.make_async_copy(src_ref.at[ridx_v.at[0]], gbuf.at[b],
                              gsem).wait()

    def _scale(jl, b):
        def _grp(g, c2):
            nv = ew_v[jl, pl.ds(g * 16, 16)]
            for r in range(16):
                ns = nv[r]
                for v in range(D // 16):
                    sl = pl.ds(v * 16, 16)
                    gbuf[b, g * 16 + r, sl] = gbuf[b, g * 16 + r, sl] * ns
            return c2
        lax.fori_loop(0, CHUNK // 16, _grp, 0)

    for st in range(CPT // STG):
        ssl = pl.ds(st * STG, STG)
        pltpu.sync_copy(ridx_hbm.at[w, ssl], ridx_v)
        pltpu.sync_copy(cidx_hbm.at[w, ssl], cidx_v)
        pltpu.sync_copy(ew_hbm.at[w, ssl], ew_v)
        _start_gather(0, 0)
        _start_gather(1, 1)

        def _pair(jj, carry):
            for b in range(2):
                jl = 2 * jj + b
                _wait_gather(b)
                _scale(jl, b)
                pltpu.sync_copy(gbuf.at[b], acc.at[cidx_v.at[jl]], add=True)

                @pl.when(jl + 2 < STG)
                def _():
                    _start_gather(jl + 2, b)
            return carry
        lax.fori_loop(0, STG // 2, _pair, 0)
    plsc.subcore_barrier()

    pltpu.sync_copy(acc.at[pl.ds(base, RPT)],
                    out_ref.at[pl.ds(c * N_PAD + base, RPT)])


_spmm = functools.partial(
    pl.kernel,
    out_type=jax.ShapeDtypeStruct((NCORE * N_PAD, D), jnp.float32),
    mesh=_mesh,
    scratch_types=[
        pltpu.VMEM_SHARED((N_PAD, D), jnp.float32),
        pltpu.VMEM((STG, CHUNK), jnp.int32),
        pltpu.VMEM((STG, CHUNK), jnp.int32),
        pltpu.VMEM((STG, CHUNK), jnp.float32),
        pltpu.VMEM((2, CHUNK, D), jnp.float32),
        pltpu.SemaphoreType.DMA,
    ],
)(_spmm_body)


NB = 10
BN = N // NB


def _dis_body(parts_ref, o_ref):
    deg = parts_ref[:N_PAD, 0:1] + parts_ref[N_PAD:, 0:1]
    pos = deg > 0
    dis = jnp.where(pos, lax.rsqrt(deg), 0.0)
    o_ref[0] = dis
    o_ref[1] = jnp.where(pos, 1.0 / deg, 0.0)
    o_ref[2] = jnp.where(pos, jnp.sqrt(deg), 0.0)


def _dis_tc(parts):
    return pl.pallas_call(
        _dis_body,
        out_shape=jax.ShapeDtypeStruct((3, N_PAD, 1), jnp.float32),
    )(parts)


def _scale_body(h_ref, d_ref, o_ref):
    o_ref[...] = h_ref[...] * d_ref[...]


def _scale_tc(h, d):
    blk = lambda i: (i, 0)
    return pl.pallas_call(
        _scale_body,
        grid=(NB,),
        in_specs=[pl.BlockSpec((BN, D), blk), pl.BlockSpec((BN, 1), blk)],
        out_specs=pl.BlockSpec((BN, D), blk),
        out_shape=jax.ShapeDtypeStruct((N_PAD, D), jnp.float32),
    )(h, d)


def _combine_body(pa, pb, d2_ref, o_ref):
    o_ref[...] = (pa[0] + pb[0]) * d2_ref[...]


def _combine(p, dis2):
    blk = lambda i: (i, 0)
    return pl.pallas_call(
        _combine_body,
        grid=(NB,),
        in_specs=[pl.BlockSpec((1, BN, D), lambda i: (0, i, 0)),
                  pl.BlockSpec((1, BN, D), lambda i: (1, i, 0)),
                  pl.BlockSpec((BN, 1), blk)],
        out_specs=pl.BlockSpec((BN, D), blk),
        out_shape=jax.ShapeDtypeStruct((N_PAD, D), jnp.float32),
    )(p, p, dis2)


def _layer_body(h_ref, s1_ref, s2_ref, t3a, t3b, dis_ref, idis_ref,
                W_ref, b_ref, o_ref, so_ref):
    idis = idis_ref[...]
    p1 = s1_ref[...] * idis
    p2 = s2_ref[...] * idis
    p3 = (t3a[0] + t3b[0]) * dis_ref[...]
    acc = jnp.dot(h_ref[...], W_ref[0], preferred_element_type=jnp.float32)
    acc += jnp.dot(p1, W_ref[1], preferred_element_type=jnp.float32)
    acc += jnp.dot(p2, W_ref[2], preferred_element_type=jnp.float32)
    acc += jnp.dot(p3, W_ref[3], preferred_element_type=jnp.float32)
    out = jnp.maximum(acc + b_ref[...], 0.0)
    o_ref[...] = out
    so_ref[...] = out * dis_ref[...]


def _layer_matmul(h, s1, s2, t3, dis, idis, W, bsum):
    blk = lambda i: (i, 0)
    out_sds = jax.ShapeDtypeStruct((N_PAD, D), jnp.float32)
    return pl.pallas_call(
        _layer_body,
        grid=(NB,),
        in_specs=[pl.BlockSpec((BN, D), blk),
                  pl.BlockSpec((BN, D), blk),
                  pl.BlockSpec((BN, D), blk),
                  pl.BlockSpec((1, BN, D), lambda i: (0, i, 0)),
                  pl.BlockSpec((1, BN, D), lambda i: (1, i, 0)),
                  pl.BlockSpec((BN, 1), blk),
                  pl.BlockSpec((BN, 1), blk),
                  pl.BlockSpec((K + 1, D, D), lambda i: (0, 0, 0)),
                  pl.BlockSpec((1, D), lambda i: (0, 0))],
        out_specs=[pl.BlockSpec((BN, D), blk), pl.BlockSpec((BN, D), blk)],
        out_shape=[out_sds, out_sds],
    )(h, s1, s2, t3, t3, dis, idis, W, bsum)


def _head_body(h_ref, batch_ref, l1w_ref, l1b_ref, l2w_ref, l2b_ref, o_ref):
    h = h_ref[...][:N]
    batch = batch_ref[...]
    gids = lax.broadcasted_iota(jnp.int32, (G, N), 0)
    sel = (gids == batch).astype(jnp.float32)
    sums = jnp.dot(sel, h, preferred_element_type=jnp.float32)
    cnt = jnp.sum(sel, axis=1, keepdims=True)
    pooled = sums / jnp.maximum(cnt, 1.0)
    y = jnp.dot(pooled, l1w_ref[...], preferred_element_type=jnp.float32)
    y = y + l1b_ref[...]
    y = jnp.dot(y, l2w_ref[...], preferred_element_type=jnp.float32)
    y = y + l2b_ref[...]
    o_ref[...] = jax.nn.sigmoid(y)


def _head(h, batch, lin1_W, lin1_b, lin2_W, lin2_b):
    return pl.pallas_call(
        _head_body,
        out_shape=jax.ShapeDtypeStruct((G, 1), jnp.float32),
    )(h, batch.reshape(1, N), lin1_W, lin1_b.reshape(1, D),
      lin2_W, lin2_b.reshape(1, 1))


def _prep_edges(arr):
    a = jnp.zeros((NW, EPT), arr.dtype)
    a = a.at[:, :EPW].set(arr.reshape(NW, EPW))
    return a.reshape(NW, CPT, CHUNK)


def kernel(x, edge_index, edge_weight, batch, conv_W, conv_b, lin1_W, lin1_b,
           lin2_W, lin2_b):
    row, col = edge_index[0], edge_index[1]
    ridx = _prep_edges(row)
    cidx = _prep_edges(col)
    ew = _prep_edges(edge_weight)

    ones = jnp.ones((N_PAD, D), jnp.float32)
    dparts = _spmm(ones, ridx, cidx, ew)
    dd = _dis_tc(dparts)
    dis, dis2, idis = dd[0], dd[1], dd[2]

    bsum = jnp.sum(conv_b, axis=1).reshape(L, 1, D)

    h = jnp.zeros((N_PAD, D), jnp.float32).at[:N].set(x)
    s0 = _scale_tc(h, dis)
    for i in range(L):
        t1 = _spmm(s0, ridx, cidx, ew).reshape(NCORE, N_PAD, D)
        s1 = _combine(t1, dis2)
        t2 = _spmm(s1, ridx, cidx, ew).reshape(NCORE, N_PAD, D)
        s2 = _combine(t2, dis2)
        t3 = _spmm(s2, ridx, cidx, ew).reshape(NCORE, N_PAD, D)
        h, s0 = _layer_matmul(h, s1, s2, t3, dis, idis, conv_W[i], bsum[i])

    return _head(h, batch, lin1_W, lin1_b, lin2_W, lin2_b)


# speedup vs baseline: 4.0485x; 1.0708x over previous
import functools
import jax
import jax.numpy as jnp
from jax import lax
from jax.experimental import pallas as pl
from jax.experimental.pallas import tpu as pltpu
from jax.experimental.pallas import tpu_sc as plsc

N = 10000
E = 320000
D = 128
K = 3
L = 5
G = 64

NSUB = 16
NCORE = 2
NW = NCORE * NSUB
CHUNK = 128
EPW = E // NW
CPT = 80
EPT = CPT * CHUNK
STG = 16
N_PAD = 10240
RPT = N_PAD // NSUB
ZR = 64

_mesh = plsc.VectorSubcoreMesh(core_axis_name="c", subcore_axis_name="s")


def _spmm_body(src_ref, ridx_hbm, cidx_hbm, ew_hbm, out_ref,
               acc, ridx_v, cidx_v, ew_v, gbuf, gsem):
    c = lax.axis_index("c")
    s = lax.axis_index("s")
    w = c * NSUB + s

    zero16 = jnp.zeros((16,), jnp.float32)

    def _z(i, carry):
        r = i // (D // 16)
        v = i % (D // 16)
        gbuf[0, r, pl.ds(v * 16, 16)] = zero16
        return carry
    lax.fori_loop(0, ZR * (D // 16), _z, 0)

    base = s * RPT
    for i in range(RPT // ZR):
        pltpu.sync_copy(gbuf.at[0, pl.ds(0, ZR)],
                        acc.at[pl.ds(base + i * ZR, ZR)])
    plsc.subcore_barrier()

    def _start_gather(jl, b):
        pltpu.async_copy(src_ref.at[ridx_v.at[jl]], gbuf.at[b], gsem)

    def _wait_gather(b):
        pltpu.make_async_copy(src_ref.at[ridx_v.at[0]], gbuf.at[b],
                              gsem).wait()

    def _scale(jl, b):
        @functools.partial(plsc.parallel_loop, 0, CHUNK // 16, unroll=2)
        def _grp(g):
            nv = ew_v[jl, pl.ds(g * 16, 16)]
            for r in range(16):
                ns = nv[r]
                for v in range(D // 16):
                    sl = pl.ds(v * 16, 16)
                    gbuf[b, g * 16 + r, sl] = gbuf[b, g * 16 + r, sl] * ns

    for st in range(CPT // STG):
        ssl = pl.ds(st * STG, STG)
        pltpu.sync_copy(ridx_hbm.at[w, ssl], ridx_v)
        pltpu.sync_copy(cidx_hbm.at[w, ssl], cidx_v)
        pltpu.sync_copy(ew_hbm.at[w, ssl], ew_v)
        _start_gather(0, 0)
        _start_gather(1, 1)

        def _pair(jj, carry):
            for b in range(2):
                jl = 2 * jj + b
                _wait_gather(b)
                _scale(jl, b)
                pltpu.sync_copy(gbuf.at[b], acc.at[cidx_v.at[jl]], add=True)

                @pl.when(jl + 2 < STG)
                def _():
                    _start_gather(jl + 2, b)
            return carry
        lax.fori_loop(0, STG // 2, _pair, 0)
    plsc.subcore_barrier()

    pltpu.sync_copy(acc.at[pl.ds(base, RPT)],
                    out_ref.at[pl.ds(c * N_PAD + base, RPT)])


_spmm = functools.partial(
    pl.kernel,
    out_type=jax.ShapeDtypeStruct((NCORE * N_PAD, D), jnp.float32),
    mesh=_mesh,
    scratch_types=[
        pltpu.VMEM_SHARED((N_PAD, D), jnp.float32),
        pltpu.VMEM((STG, CHUNK), jnp.int32),
        pltpu.VMEM((STG, CHUNK), jnp.int32),
        pltpu.VMEM((STG, CHUNK), jnp.float32),
        pltpu.VMEM((2, CHUNK, D), jnp.float32),
        pltpu.SemaphoreType.DMA,
    ],
)(_spmm_body)


NB = 10
BN = N // NB


def _dis_body(parts_ref, o_ref):
    deg = parts_ref[:N_PAD, 0:1] + parts_ref[N_PAD:, 0:1]
    pos = deg > 0
    dis = jnp.where(pos, lax.rsqrt(deg), 0.0)
    o_ref[0] = dis
    o_ref[1] = jnp.where(pos, 1.0 / deg, 0.0)
    o_ref[2] = jnp.where(pos, jnp.sqrt(deg), 0.0)


def _dis_tc(parts):
    return pl.pallas_call(
        _dis_body,
        out_shape=jax.ShapeDtypeStruct((3, N_PAD, 1), jnp.float32),
    )(parts)


def _scale_body(h_ref, d_ref, o_ref):
    o_ref[...] = h_ref[...] * d_ref[...]


def _scale_tc(h, d):
    blk = lambda i: (i, 0)
    return pl.pallas_call(
        _scale_body,
        grid=(NB,),
        in_specs=[pl.BlockSpec((BN, D), blk), pl.BlockSpec((BN, 1), blk)],
        out_specs=pl.BlockSpec((BN, D), blk),
        out_shape=jax.ShapeDtypeStruct((N_PAD, D), jnp.float32),
    )(h, d)


def _combine_body(pa, pb, d2_ref, o_ref):
    o_ref[...] = (pa[0] + pb[0]) * d2_ref[...]


def _combine(p, dis2):
    blk = lambda i: (i, 0)
    return pl.pallas_call(
        _combine_body,
        grid=(NB,),
        in_specs=[pl.BlockSpec((1, BN, D), lambda i: (0, i, 0)),
                  pl.BlockSpec((1, BN, D), lambda i: (1, i, 0)),
                  pl.BlockSpec((BN, 1), blk)],
        out_specs=pl.BlockSpec((BN, D), blk),
        out_shape=jax.ShapeDtypeStruct((N_PAD, D), jnp.float32),
    )(p, p, dis2)


def _layer_body(h_ref, s1_ref, s2_ref, t3a, t3b, dis_ref, idis_ref,
                W_ref, b_ref, o_ref, so_ref):
    idis = idis_ref[...]
    p1 = s1_ref[...] * idis
    p2 = s2_ref[...] * idis
    p3 = (t3a[0] + t3b[0]) * dis_ref[...]
    acc = jnp.dot(h_ref[...], W_ref[0], preferred_element_type=jnp.float32)
    acc += jnp.dot(p1, W_ref[1], preferred_element_type=jnp.float32)
    acc += jnp.dot(p2, W_ref[2], preferred_element_type=jnp.float32)
    acc += jnp.dot(p3, W_ref[3], preferred_element_type=jnp.float32)
    out = jnp.maximum(acc + b_ref[...], 0.0)
    o_ref[...] = out
    so_ref[...] = out * dis_ref[...]


def _layer_matmul(h, s1, s2, t3, dis, idis, W, bsum):
    blk = lambda i: (i, 0)
    out_sds = jax.ShapeDtypeStruct((N_PAD, D), jnp.float32)
    return pl.pallas_call(
        _layer_body,
        grid=(NB,),
        in_specs=[pl.BlockSpec((BN, D), blk),
                  pl.BlockSpec((BN, D), blk),
                  pl.BlockSpec((BN, D), blk),
                  pl.BlockSpec((1, BN, D), lambda i: (0, i, 0)),
                  pl.BlockSpec((1, BN, D), lambda i: (1, i, 0)),
                  pl.BlockSpec((BN, 1), blk),
                  pl.BlockSpec((BN, 1), blk),
                  pl.BlockSpec((K + 1, D, D), lambda i: (0, 0, 0)),
                  pl.BlockSpec((1, D), lambda i: (0, 0))],
        out_specs=[pl.BlockSpec((BN, D), blk), pl.BlockSpec((BN, D), blk)],
        out_shape=[out_sds, out_sds],
    )(h, s1, s2, t3, t3, dis, idis, W, bsum)


def _head_body(h_ref, batch_ref, l1w_ref, l1b_ref, l2w_ref, l2b_ref, o_ref):
    h = h_ref[...][:N]
    batch = batch_ref[...]
    gids = lax.broadcasted_iota(jnp.int32, (G, N), 0)
    sel = (gids == batch).astype(jnp.float32)
    sums = jnp.dot(sel, h, preferred_element_type=jnp.float32)
    cnt = jnp.sum(sel, axis=1, keepdims=True)
    pooled = sums / jnp.maximum(cnt, 1.0)
    y = jnp.dot(pooled, l1w_ref[...], preferred_element_type=jnp.float32)
    y = y + l1b_ref[...]
    y = jnp.dot(y, l2w_ref[...], preferred_element_type=jnp.float32)
    y = y + l2b_ref[...]
    o_ref[...] = jax.nn.sigmoid(y)


def _head(h, batch, lin1_W, lin1_b, lin2_W, lin2_b):
    return pl.pallas_call(
        _head_body,
        out_shape=jax.ShapeDtypeStruct((G, 1), jnp.float32),
    )(h, batch.reshape(1, N), lin1_W, lin1_b.reshape(1, D),
      lin2_W, lin2_b.reshape(1, 1))


def _prep_edges(arr):
    a = jnp.zeros((NW, EPT), arr.dtype)
    a = a.at[:, :EPW].set(arr.reshape(NW, EPW))
    return a.reshape(NW, CPT, CHUNK)


def kernel(x, edge_index, edge_weight, batch, conv_W, conv_b, lin1_W, lin1_b,
           lin2_W, lin2_b):
    row, col = edge_index[0], edge_index[1]
    ridx = _prep_edges(row)
    cidx = _prep_edges(col)
    ew = _prep_edges(edge_weight)

    ones = jnp.ones((N_PAD, D), jnp.float32)
    dparts = _spmm(ones, ridx, cidx, ew)
    dd = _dis_tc(dparts)
    dis, dis2, idis = dd[0], dd[1], dd[2]

    bsum = jnp.sum(conv_b, axis=1).reshape(L, 1, D)

    h = jnp.zeros((N_PAD, D), jnp.float32).at[:N].set(x)
    s0 = _scale_tc(h, dis)
    for i in range(L):
        t1 = _spmm(s0, ridx, cidx, ew).reshape(NCORE, N_PAD, D)
        s1 = _combine(t1, dis2)
        t2 = _spmm(s1, ridx, cidx, ew).reshape(NCORE, N_PAD, D)
        s2 = _combine(t2, dis2)
        t3 = _spmm(s2, ridx, cidx, ew).reshape(NCORE, N_PAD, D)
        h, s0 = _layer_matmul(h, s1, s2, t3, dis, idis, conv_W[i], bsum[i])

    return _head(h, batch, lin1_W, lin1_b, lin2_W, lin2_b)


# speedup vs baseline: 4.2604x; 1.0523x over previous
import functools
import jax
import jax.numpy as jnp
from jax import lax
from jax.experimental import pallas as pl
from jax.experimental.pallas import tpu as pltpu
from jax.experimental.pallas import tpu_sc as plsc

N = 10000
E = 320000
D = 128
K = 3
L = 5
G = 64

NSUB = 16
NCORE = 2
NW = NCORE * NSUB
CHUNK = 128
EPW = E // NW
CPT = 80
EPT = CPT * CHUNK
STG = 16
N_PAD = 10240
RPT = N_PAD // NSUB
ZR = 64

_mesh = plsc.VectorSubcoreMesh(core_axis_name="c", subcore_axis_name="s")


def _spmm_body(src_ref, ridx_hbm, cidx_hbm, ew_hbm, out_ref,
               acc, ridx_v, cidx_v, ew_v, gbuf, gsem):
    c = lax.axis_index("c")
    s = lax.axis_index("s")
    w = c * NSUB + s

    zero16 = jnp.zeros((16,), jnp.float32)

    def _z(i, carry):
        r = i // (D // 16)
        v = i % (D // 16)
        gbuf[0, r, pl.ds(v * 16, 16)] = zero16
        return carry
    lax.fori_loop(0, ZR * (D // 16), _z, 0)

    base = s * RPT
    for i in range(RPT // ZR):
        pltpu.sync_copy(gbuf.at[0, pl.ds(0, ZR)],
                        acc.at[pl.ds(base + i * ZR, ZR)])
    plsc.subcore_barrier()

    def _start_gather(jl, b):
        pltpu.async_copy(src_ref.at[ridx_v.at[jl]], gbuf.at[b], gsem)

    def _wait_gather(b):
        pltpu.make_async_copy(src_ref.at[ridx_v.at[0]], gbuf.at[b],
                              gsem).wait()

    def _scale(jl, b):
        @functools.partial(plsc.parallel_loop, 0, CHUNK // 16, unroll=2)
        def _grp(g):
            nv = ew_v[jl, pl.ds(g * 16, 16)]
            for r in range(16):
                ns = nv[r]
                for v in range(D // 16):
                    sl = pl.ds(v * 16, 16)
                    gbuf[b, g * 16 + r, sl] = gbuf[b, g * 16 + r, sl] * ns

    for st in range(CPT // STG):
        ssl = pl.ds(st * STG, STG)
        pltpu.sync_copy(ridx_hbm.at[w, ssl], ridx_v)
        pltpu.sync_copy(cidx_hbm.at[w, ssl], cidx_v)
        pltpu.sync_copy(ew_hbm.at[w, ssl], ew_v)
        _start_gather(0, 0)
        _start_gather(1, 1)

        def _pair(jj, carry):
            for b in range(2):
                jl = 2 * jj + b
                _wait_gather(b)
                _scale(jl, b)
                pltpu.sync_copy(gbuf.at[b], acc.at[cidx_v.at[jl]], add=True)

                @pl.when(jl + 2 < STG)
                def _():
                    _start_gather(jl + 2, b)
            return carry
        lax.fori_loop(0, STG // 2, _pair, 0)
    plsc.subcore_barrier()

    pltpu.sync_copy(acc.at[pl.ds(base, RPT)],
                    out_ref.at[pl.ds(c * N_PAD + base, RPT)])


_spmm = functools.partial(
    pl.kernel,
    out_type=jax.ShapeDtypeStruct((NCORE * N_PAD, D), jnp.float32),
    mesh=_mesh,
    scratch_types=[
        pltpu.VMEM_SHARED((N_PAD, D), jnp.float32),
        pltpu.VMEM((STG, CHUNK), jnp.int32),
        pltpu.VMEM((STG, CHUNK), jnp.int32),
        pltpu.VMEM((STG, CHUNK), jnp.float32),
        pltpu.VMEM((2, CHUNK, D), jnp.float32),
        pltpu.SemaphoreType.DMA,
    ],
)(_spmm_body)


def _degp_body(cidx_hbm, ew_hbm, out_ref, acc, cidx_v, ew_v, gbuf):
    c = lax.axis_index("c")
    s = lax.axis_index("s")
    w = c * NSUB + s

    zero16 = jnp.zeros((16,), jnp.float32)

    def _z(i, carry):
        r = i // (D // 16)
        v = i % (D // 16)
        gbuf[r, pl.ds(v * 16, 16)] = zero16
        return carry
    lax.fori_loop(0, ZR * (D // 16), _z, 0)

    base = s * RPT
    for i in range(RPT // ZR):
        pltpu.sync_copy(gbuf.at[pl.ds(0, ZR)],
                        acc.at[pl.ds(base + i * ZR, ZR)])
    plsc.subcore_barrier()

    for st in range(CPT // STG):
        ssl = pl.ds(st * STG, STG)
        pltpu.sync_copy(cidx_hbm.at[w, ssl], cidx_v)
        pltpu.sync_copy(ew_hbm.at[w, ssl], ew_v)

        def _chunk(jl, carry):
            @functools.partial(plsc.parallel_loop, 0, CHUNK // 16, unroll=2)
            def _grp(g):
                nv = ew_v[jl, pl.ds(g * 16, 16)]
                for r in range(16):
                    bc = jnp.zeros((16,), jnp.float32) + nv[r]
                    for v in range(D // 16):
                        gbuf[g * 16 + r, pl.ds(v * 16, 16)] = bc
            pltpu.sync_copy(gbuf, acc.at[cidx_v.at[jl]], add=True)
            return carry
        lax.fori_loop(0, STG, _chunk, 0)
    plsc.subcore_barrier()

    pltpu.sync_copy(acc.at[pl.ds(base, RPT)],
                    out_ref.at[pl.ds(c * N_PAD + base, RPT)])


_degp = functools.partial(
    pl.kernel,
    out_type=jax.ShapeDtypeStruct((NCORE * N_PAD, D), jnp.float32),
    mesh=_mesh,
    scratch_types=[
        pltpu.VMEM_SHARED((N_PAD, D), jnp.float32),
        pltpu.VMEM((STG, CHUNK), jnp.int32),
        pltpu.VMEM((STG, CHUNK), jnp.float32),
        pltpu.VMEM((CHUNK, D), jnp.float32),
    ],
)(_degp_body)


NB = 10
BN = N // NB


def _dis_body(parts_ref, o_ref):
    deg = parts_ref[:N_PAD, 0:1] + parts_ref[N_PAD:, 0:1]
    pos = deg > 0
    dis = jnp.where(pos, lax.rsqrt(deg), 0.0)
    o_ref[0] = dis
    o_ref[1] = jnp.where(pos, 1.0 / deg, 0.0)
    o_ref[2] = jnp.where(pos, jnp.sqrt(deg), 0.0)


def _dis_tc(parts):
    return pl.pallas_call(
        _dis_body,
        out_shape=jax.ShapeDtypeStruct((3, N_PAD, 1), jnp.float32),
    )(parts)


def _scale_body(h_ref, d_ref, o_ref):
    o_ref[...] = h_ref[...] * d_ref[...]


def _scale_tc(h, d):
    blk = lambda i: (i, 0)
    return pl.pallas_call(
        _scale_body,
        grid=(NB,),
        in_specs=[pl.BlockSpec((BN, D), blk), pl.BlockSpec((BN, 1), blk)],
        out_specs=pl.BlockSpec((BN, D), blk),
        out_shape=jax.ShapeDtypeStruct((N_PAD, D), jnp.float32),
    )(h, d)


def _combine_body(pa, pb, d2_ref, o_ref):
    o_ref[...] = (pa[0] + pb[0]) * d2_ref[...]


def _combine(p, dis2):
    blk = lambda i: (i, 0)
    return pl.pallas_call(
        _combine_body,
        grid=(NB,),
        in_specs=[pl.BlockSpec((1, BN, D), lambda i: (0, i, 0)),
                  pl.BlockSpec((1, BN, D), lambda i: (1, i, 0)),
                  pl.BlockSpec((BN, 1), blk)],
        out_specs=pl.BlockSpec((BN, D), blk),
        out_shape=jax.ShapeDtypeStruct((N_PAD, D), jnp.float32),
    )(p, p, dis2)


def _layer_body(h_ref, s1_ref, s2_ref, t3a, t3b, dis_ref, idis_ref,
                W_ref, b_ref, o_ref, so_ref):
    idis = idis_ref[...]
    p1 = s1_ref[...] * idis
    p2 = s2_ref[...] * idis
    p3 = (t3a[0] + t3b[0]) * dis_ref[...]
    acc = jnp.dot(h_ref[...], W_ref[0], preferred_element_type=jnp.float32)
    acc += jnp.dot(p1, W_ref[1], preferred_element_type=jnp.float32)
    acc += jnp.dot(p2, W_ref[2], preferred_element_type=jnp.float32)
    acc += jnp.dot(p3, W_ref[3], preferred_element_type=jnp.float32)
    out = jnp.maximum(acc + b_ref[...], 0.0)
    o_ref[...] = out
    so_ref[...] = out * dis_ref[...]


def _layer_matmul(h, s1, s2, t3, dis, idis, W, bsum):
    blk = lambda i: (i, 0)
    out_sds = jax.ShapeDtypeStruct((N_PAD, D), jnp.float32)
    return pl.pallas_call(
        _layer_body,
        grid=(NB,),
        in_specs=[pl.BlockSpec((BN, D), blk),
                  pl.BlockSpec((BN, D), blk),
                  pl.BlockSpec((BN, D), blk),
                  pl.BlockSpec((1, BN, D), lambda i: (0, i, 0)),
                  pl.BlockSpec((1, BN, D), lambda i: (1, i, 0)),
                  pl.BlockSpec((BN, 1), blk),
                  pl.BlockSpec((BN, 1), blk),
                  pl.BlockSpec((K + 1, D, D), lambda i: (0, 0, 0)),
                  pl.BlockSpec((1, D), lambda i: (0, 0))],
        out_specs=[pl.BlockSpec((BN, D), blk), pl.BlockSpec((BN, D), blk)],
        out_shape=[out_sds, out_sds],
    )(h, s1, s2, t3, t3, dis, idis, W, bsum)


def _head_body(h_ref, batch_ref, l1w_ref, l1b_ref, l2w_ref, l2b_ref, o_ref):
    h = h_ref[...][:N]
    batch = batch_ref[...]
    gids = lax.broadcasted_iota(jnp.int32, (G, N), 0)
    sel = (gids == batch).astype(jnp.float32)
    sums = jnp.dot(sel, h, preferred_element_type=jnp.float32)
    cnt = jnp.sum(sel, axis=1, keepdims=True)
    pooled = sums / jnp.maximum(cnt, 1.0)
    y = jnp.dot(pooled, l1w_ref[...], preferred_element_type=jnp.float32)
    y = y + l1b_ref[...]
    y = jnp.dot(y, l2w_ref[...], preferred_element_type=jnp.float32)
    y = y + l2b_ref[...]
    o_ref[...] = jax.nn.sigmoid(y)


def _head(h, batch, lin1_W, lin1_b, lin2_W, lin2_b):
    return pl.pallas_call(
        _head_body,
        out_shape=jax.ShapeDtypeStruct((G, 1), jnp.float32),
    )(h, batch.reshape(1, N), lin1_W, lin1_b.reshape(1, D),
      lin2_W, lin2_b.reshape(1, 1))


def _prep_edges(arr):
    a = jnp.zeros((NW, EPT), arr.dtype)
    a = a.at[:, :EPW].set(arr.reshape(NW, EPW))
    return a.reshape(NW, CPT, CHUNK)


def kernel(x, edge_index, edge_weight, batch, conv_W, conv_b, lin1_W, lin1_b,
           lin2_W, lin2_b):
    row, col = edge_index[0], edge_index[1]
    ridx = _prep_edges(row)
    cidx = _prep_edges(col)
    ew = _prep_edges(edge_weight)

    dparts = _degp(cidx, ew)
    dd = _dis_tc(dparts)
    dis, dis2, idis = dd[0], dd[1], dd[2]

    bsum = jnp.sum(conv_b, axis=1).reshape(L, 1, D)

    h = jnp.zeros((N_PAD, D), jnp.float32).at[:N].set(x)
    s0 = _scale_tc(h, dis)
    for i in range(L):
        t1 = _spmm(s0, ridx, cidx, ew).reshape(NCORE, N_PAD, D)
        s1 = _combine(t1, dis2)
        t2 = _spmm(s1, ridx, cidx, ew).reshape(NCORE, N_PAD, D)
        s2 = _combine(t2, dis2)
        t3 = _spmm(s2, ridx, cidx, ew).reshape(NCORE, N_PAD, D)
        h, s0 = _layer_matmul(h, s1, s2, t3, dis, idis, conv_W[i], bsum[i])

    return _head(h, batch, lin1_W, lin1_b, lin2_W, lin2_b)
